# baseline (speedup 1.0000x reference)
import functools

import jax
import jax.numpy as jnp
from jax import lax
from jax.experimental import pallas as pl
from jax.experimental.pallas import tpu as pltpu
from jax.experimental.pallas import tpu_sc as plsc

B, N = 32, 36864
BN = B * N
KBLK = N // 128
RPB = KBLK * 4
ROWS = B * RPB

TC_FRAC = 384
TC_K = TC_FRAC // 4
OBJ_RPB = KBLK


KT = 3
KSUB = TC_K // KT
OBJ_BLK = 9216 // (4 * KT)


def _tc_body(op_ref, gt_ref, gtm_ref, *refs):
    bp_refs = refs[:8]
    bg_refs = refs[8:16]
    bce_ref, box_ref = refs[16:]
    bb = pl.program_id(0)
    t = pl.program_id(1)

    x = op_ref[...]
    tt = gt_ref[...]
    per = jnp.maximum(x, 0.0) - x * tt + jnp.log1p(jnp.exp(-jnp.abs(x)))
    s_bce = jnp.sum(per)

    maskc = (gtm_ref[...] == 1.0).astype(jnp.float32)
    fk = jax.lax.broadcasted_iota(jnp.int32, (KSUB, KSUB * 4), 0)
    fr = jax.lax.broadcasted_iota(jnp.int32, (KSUB, KSUB * 4), 1)
    fold_m = (fr // 4 == fk).astype(jnp.float32)
    dk = jax.lax.broadcasted_iota(jnp.int32, (KSUB, KSUB * 8), 0)
    dq = jax.lax.broadcasted_iota(jnp.int32, (KSUB, KSUB * 8), 1)
    s_box = jnp.zeros((), jnp.float32)
    for j in range(8):
        d = bp_refs[j][...] - bg_refs[j][...]
        a = jnp.abs(d)
        c = jnp.minimum(a, 1.0)
        per2 = c * (a + a - c)
        folded = jax.lax.dot(fold_m, per2,
                             preferred_element_type=jnp.float32)
        sel_j = (dq == 8 * dk + j).astype(jnp.float32)
        mask_j = jax.lax.dot(sel_j, maskc,
                             preferred_element_type=jnp.float32)
        s_box = s_box + jnp.sum(folded * mask_j)

    @pl.when(jnp.logical_and(bb == 0, t == 0))
    def _():
        bce_ref[...] = jnp.zeros_like(bce_ref)
        box_ref[...] = jnp.zeros_like(box_ref)

    bce_ref[...] += jnp.broadcast_to(s_bce, (1, 1))
    box_ref[...] += jnp.broadcast_to(s_box, (1, 1))


def _tc_main(op4, gt4o, bpv, bgv):
    bbox_specs = [
        pl.BlockSpec((KSUB * 4, 128),
                     (lambda j: lambda bb, t: ((8 * bb + j) * 9 + t, 0))(j))
        for j in range(8)
    ]
    return pl.pallas_call(
        _tc_body,
        grid=(4, KT),
        in_specs=[
            pl.BlockSpec((OBJ_BLK, 128), lambda bb, t: (KT * bb + t, 0)),
            pl.BlockSpec((OBJ_BLK, 128), lambda bb, t: (KT * bb + t, 0)),
            pl.BlockSpec((KSUB * 8, 128), lambda bb, t: (9 * bb + t, 0)),
        ] + bbox_specs + bbox_specs,
        out_specs=[pl.BlockSpec((1, 1), lambda bb, t: (0, 0))] * 2,
        out_shape=[jax.ShapeDtypeStruct((1, 1), jnp.float32)] * 2,
    )(op4, gt4o, gt4o, *([bpv] * 8), *([bgv] * 8))


NW = 32
CHUNK_ROWS = 96
SC_ROWS = RPB - TC_FRAC
N_CHUNKS = SC_ROWS // CHUNK_ROWS
CHUNK_K = CHUNK_ROWS // 4
SC_K = KBLK - TC_K

_sc_mesh = plsc.VectorSubcoreMesh(core_axis_name="c", subcore_axis_name="s")


@functools.partial(
    pl.kernel,
    mesh=_sc_mesh,
    out_type=jax.ShapeDtypeStruct((NW, 16), jnp.float32),
    scratch_types=[
        pltpu.VMEM((SC_K, 128), jnp.float32),
        pltpu.VMEM((CHUNK_ROWS, 128), jnp.float32),
        pltpu.VMEM((CHUNK_ROWS, 128), jnp.float32),
        pltpu.VMEM((CHUNK_ROWS, 128), jnp.float32),
        pltpu.VMEM((CHUNK_ROWS, 128), jnp.float32),
        pltpu.VMEM((16,), jnp.float32),
        pltpu.SemaphoreType.DMA,
        pltpu.SemaphoreType.DMA,
        pltpu.SemaphoreType.DMA,
        pltpu.SemaphoreType.DMA,
    ],
)
def _sc_bbox(bp_hbm, bg_hbm, gt4_hbm, out_hbm,
             mask_v, bp_a, bp_b, bg_a, bg_b, acc_v,
             sem_a, sem_b, sem_a2, sem_b2):
    cid = lax.axis_index("c")
    sid = lax.axis_index("s")
    w = sid * 2 + cid
    bb = w // 8
    bi = w % 8

    pltpu.sync_copy(gt4_hbm.at[bb, pl.ds(TC_K, SC_K), bi, :], mask_v)

    row0 = w * RPB + TC_FRAC

    def start(step, bp_buf, bg_buf, sp, sg):
        r = row0 + step * CHUNK_ROWS
        pltpu.async_copy(bp_hbm.at[pl.ds(r, CHUNK_ROWS), :], bp_buf, sp).start()
        pltpu.async_copy(bg_hbm.at[pl.ds(r, CHUNK_ROWS), :], bg_buf, sg).start()

    def wait(bp_buf, bg_buf, sp, sg):
        pltpu.make_async_copy(bp_hbm.at[pl.ds(0, CHUNK_ROWS), :],
                              bp_buf, sp).wait()
        pltpu.make_async_copy(bg_hbm.at[pl.ds(0, CHUNK_ROWS), :],
                              bg_buf, sg).wait()

    def chunk_sum(step, bp_buf, bg_buf, acc0):
        k0 = step * CHUNK_K

        def kk_body(kk, acc):
            def a_body(ai, acc2):
                a0 = ai * 16
                m = mask_v[k0 + kk, pl.ds(a0, 16)]

                def c_body(cc, acc3):
                    r = kk * 4 + cc
                    p = bp_buf[r, pl.ds(a0, 16)]
                    g = bg_buf[r, pl.ds(a0, 16)]
                    d = p - g
                    a = jnp.abs(d)
                    cl = jnp.minimum(a, 1.0)
                    per2 = cl * (a + a - cl)
                    return acc3 + per2 * m

                return lax.fori_loop(0, 4, c_body, acc2)

            return lax.fori_loop(0, 8, a_body, acc)

        return lax.fori_loop(0, CHUNK_K, kk_body, acc0)

    start(0, bp_a, bg_a, sem_a, sem_a2)

    def pair_body(i, total):
        start(2 * i + 1, bp_b, bg_b, sem_b, sem_b2)
        wait(bp_a, bg_a, sem_a, sem_a2)
        total = chunk_sum(2 * i, bp_a, bg_a, total)

        @pl.when(i + 1 < N_CHUNKS // 2)
        def _():
            start(2 * i + 2, bp_a, bg_a, sem_a, sem_a2)

        wait(bp_b, bg_b, sem_b, sem_b2)
        return chunk_sum(2 * i + 1, bp_b, bg_b, total)

    total = lax.fori_loop(0, N_CHUNKS // 2, pair_body,
                          jnp.zeros((16,), jnp.float32))

    acc_v[...] = total
    pltpu.sync_copy(acc_v, out_hbm.at[w])


def _combine_body(lo_ref, lb_ref, bce_ref, btc_ref, part_ref,
                  o1_ref, o2_ref, o3_ref):
    o1 = lo_ref[...] * bce_ref[...] * (1.0 / BN)
    o2 = lb_ref[...] * 0.5 * (jnp.broadcast_to(jnp.sum(part_ref[...]), (1, 1))
                              + btc_ref[...])
    o1_ref[...] = o1
    o2_ref[...] = o2
    o3_ref[...] = o1 + o2


def _combine(lam_o, lam_b, bce, box_tc, parts):
    return pl.pallas_call(
        _combine_body,
        in_specs=[pl.BlockSpec((1, 1), lambda: (0, 0))] * 4
        + [pl.BlockSpec((NW, 16), lambda: (0, 0))],
        out_specs=[pl.BlockSpec((1, 1), lambda: (0, 0))] * 3,
        out_shape=[jax.ShapeDtypeStruct((1, 1), jnp.float32)] * 3,
    )(lam_o, lam_b, bce, box_tc, parts)


def kernel(objectness_pred, bbox_pred, objectness_gt, bbox_gt,
           lambda_rpn_objectness, lambda_rpn_bbox):
    bp_view = (bbox_pred.reshape(B, KBLK, 128, 4)
               .transpose(0, 1, 3, 2).reshape(ROWS, 128))
    bg_view = (bbox_gt.reshape(B, KBLK, 128, 4)
               .transpose(0, 1, 3, 2).reshape(ROWS, 128))
    gt4_view = (objectness_gt.reshape(4, 8, KBLK, 128)
                .transpose(0, 2, 1, 3))
    gt_obj = gt4_view.reshape(B * KBLK, 128)
    op_obj = (objectness_pred.reshape(4, 8, KBLK, 128)
              .transpose(0, 2, 1, 3).reshape(B * KBLK, 128))
    gt_obj = pltpu.with_memory_space_constraint(gt_obj, pltpu.MemorySpace.HBM)
    op_obj = pltpu.with_memory_space_constraint(op_obj, pltpu.MemorySpace.HBM)
    bp_view = pltpu.with_memory_space_constraint(bp_view, pltpu.MemorySpace.HBM)
    bg_view = pltpu.with_memory_space_constraint(bg_view, pltpu.MemorySpace.HBM)
    gt4_view = pltpu.with_memory_space_constraint(gt4_view,
                                                 pltpu.MemorySpace.HBM)

    parts = _sc_bbox(bp_view, bg_view, gt4_view)
    bce, box_tc = _tc_main(op_obj, gt_obj, bp_view, bg_view)
    lam_o = jnp.asarray(lambda_rpn_objectness, jnp.float32).reshape(1, 1)
    lam_b = jnp.asarray(lambda_rpn_bbox, jnp.float32).reshape(1, 1)
    o1, o2, o3 = _combine(lam_o, lam_b, bce, box_tc, parts)
    return (o1.reshape(()), o2.reshape(()), o3.reshape(()))


# speedup vs baseline: 1.0986x; 1.0986x over previous
import functools

import jax
import jax.numpy as jnp
from jax import lax
from jax.experimental import pallas as pl
from jax.experimental.pallas import tpu as pltpu
from jax.experimental.pallas import tpu_sc as plsc

B, N = 32, 36864
BN = B * N
KBLK = N // 128
RPB = KBLK * 4
ROWS = B * RPB

TC_K = 96
TC_FRAC = TC_K * 4
GBLK = 2304


def _tc_body(op_ref, gt_ref, *refs):
    bp_refs = refs[:8]
    bg_refs = refs[8:16]
    bce_ref, box_ref = refs[16:]
    bb = pl.program_id(0)

    x = op_ref[...]
    tt = gt_ref[...]
    per = jnp.maximum(x, 0.0) - x * tt + jnp.log1p(jnp.exp(-jnp.abs(x)))
    s_bce = jnp.sum(per)

    maskc = (gt_ref[0:TC_K * 8, :] == 1.0).astype(jnp.float32)
    maskt = jnp.transpose(maskc.reshape(TC_K, 8, 128), (1, 0, 2))
    fk = jax.lax.broadcasted_iota(jnp.int32, (TC_K, TC_K * 4), 0)
    fr = jax.lax.broadcasted_iota(jnp.int32, (TC_K, TC_K * 4), 1)
    fold_m = (fr // 4 == fk).astype(jnp.float32)
    s_box = jnp.zeros((), jnp.float32)
    for j in range(8):
        d = bp_refs[j][...] - bg_refs[j][...]
        a = jnp.abs(d)
        c = jnp.minimum(a, 1.0)
        per2 = c * (a + a - c)
        folded = jax.lax.dot(fold_m, per2,
                             preferred_element_type=jnp.float32)
        s_box = s_box + jnp.sum(folded * maskt[j])

    @pl.when(bb == 0)
    def _():
        bce_ref[...] = jnp.zeros_like(bce_ref)
        box_ref[...] = jnp.zeros_like(box_ref)

    bce_ref[...] += jnp.broadcast_to(s_bce, (1, 1))
    box_ref[...] += jnp.broadcast_to(s_box, (1, 1))


def _tc_main(op4, gt4o, bpv, bgv):
    bbox_specs = [
        pl.BlockSpec((TC_FRAC, 128),
                     (lambda j: lambda bb: ((8 * bb + j) * 3, 0))(j))
        for j in range(8)
    ]
    return pl.pallas_call(
        _tc_body,
        grid=(4,),
        in_specs=[
            pl.BlockSpec((GBLK, 128), lambda bb: (bb, 0)),
            pl.BlockSpec((GBLK, 128), lambda bb: (bb, 0)),
        ] + bbox_specs + bbox_specs,
        out_specs=[pl.BlockSpec((1, 1), lambda bb: (0, 0))] * 2,
        out_shape=[jax.ShapeDtypeStruct((1, 1), jnp.float32)] * 2,
    )(op4, gt4o, *([bpv] * 8), *([bgv] * 8))


NW = 32
CHUNK_ROWS = 128
SC_ROWS = RPB - TC_FRAC
N_CHUNKS = SC_ROWS // CHUNK_ROWS
SC_K = KBLK - TC_K

_sc_mesh = plsc.VectorSubcoreMesh(core_axis_name="c", subcore_axis_name="s")


@functools.partial(
    pl.kernel,
    mesh=_sc_mesh,
    out_type=jax.ShapeDtypeStruct((NW, 16), jnp.float32),
    scratch_types=[
        pltpu.VMEM((SC_K, 128), jnp.float32),
        pltpu.VMEM((2, CHUNK_ROWS, 128), jnp.float32),
        pltpu.VMEM((2, CHUNK_ROWS, 128), jnp.float32),
        pltpu.VMEM((16,), jnp.float32),
        pltpu.SemaphoreType.DMA,
        pltpu.SemaphoreType.DMA,
        pltpu.SemaphoreType.DMA,
    ],
)
def _sc_bbox(bp_hbm, bg_hbm, gt4_hbm, out_hbm,
             mask_v, bp_v, bg_v, acc_v, sem_m, sem_p, sem_g):
    cid = lax.axis_index("c")
    sid = lax.axis_index("s")
    w = sid * 2 + cid
    bb = w // 8
    bi = w % 8

    pltpu.sync_copy(gt4_hbm.at[bb, pl.ds(TC_K, SC_K), bi, :], mask_v)

    row0 = w * RPB + TC_FRAC

    def chunk_start(step, buf):
        r = row0 + step * CHUNK_ROWS
        cp = pltpu.async_copy(bp_hbm.at[pl.ds(r, CHUNK_ROWS), :],
                              bp_v.at[buf], sem_p)
        cg = pltpu.async_copy(bg_hbm.at[pl.ds(r, CHUNK_ROWS), :],
                              bg_v.at[buf], sem_g)
        return cp, cg

    def chunk_sum(step, buf):
        k0 = step * (CHUNK_ROWS // 4)

        def kk_body(kk, acc):
            def a_body(ai, acc2):
                a0 = ai * 16
                m = mask_v[k0 + kk, pl.ds(a0, 16)]

                def c_body(cc, acc3):
                    r = kk * 4 + cc
                    p = bp_v[buf, r, pl.ds(a0, 16)]
                    g = bg_v[buf, r, pl.ds(a0, 16)]
                    d = p - g
                    a = jnp.abs(d)
                    cl = jnp.minimum(a, 1.0)
                    per2 = cl * (a + a - cl)
                    return acc3 + per2 * m

                return lax.fori_loop(0, 4, c_body, acc2)

            return lax.fori_loop(0, 8, a_body, acc)

        return lax.fori_loop(0, CHUNK_ROWS // 4, kk_body,
                             jnp.zeros((16,), jnp.float32))

    total = jnp.zeros((16,), jnp.float32)
    cp, cg = chunk_start(0, 0)
    for step in range(N_CHUNKS):
        cp.wait()
        cg.wait()
        if step + 1 < N_CHUNKS:
            cp, cg = chunk_start(step + 1, (step + 1) % 2)
        total = total + chunk_sum(step, step % 2)

    acc_v[...] = total
    pltpu.sync_copy(acc_v, out_hbm.at[w])


def _combine_body(lo_ref, lb_ref, bce_ref, btc_ref, part_ref,
                  o1_ref, o2_ref, o3_ref):
    o1 = lo_ref[...] * bce_ref[...] * (1.0 / BN)
    o2 = lb_ref[...] * 0.5 * (jnp.broadcast_to(jnp.sum(part_ref[...]), (1, 1))
                              + btc_ref[...])
    o1_ref[...] = o1
    o2_ref[...] = o2
    o3_ref[...] = o1 + o2


def _combine(lam_o, lam_b, bce, box_tc, parts):
    return pl.pallas_call(
        _combine_body,
        in_specs=[pl.BlockSpec((1, 1), lambda: (0, 0))] * 4
        + [pl.BlockSpec((NW, 16), lambda: (0, 0))],
        out_specs=[pl.BlockSpec((1, 1), lambda: (0, 0))] * 3,
        out_shape=[jax.ShapeDtypeStruct((1, 1), jnp.float32)] * 3,
    )(lam_o, lam_b, bce, box_tc, parts)


def kernel(objectness_pred, bbox_pred, objectness_gt, bbox_gt,
           lambda_rpn_objectness, lambda_rpn_bbox):
    bp_view = (bbox_pred.reshape(B, KBLK, 128, 4)
               .transpose(0, 1, 3, 2).reshape(ROWS, 128))
    bg_view = (bbox_gt.reshape(B, KBLK, 128, 4)
               .transpose(0, 1, 3, 2).reshape(ROWS, 128))
    gt4_view = (objectness_gt.reshape(4, 8, KBLK, 128)
                .transpose(0, 2, 1, 3))
    gt_obj = gt4_view.reshape(B * KBLK, 128)
    op_obj = (objectness_pred.reshape(4, 8, KBLK, 128)
              .transpose(0, 2, 1, 3).reshape(B * KBLK, 128))
    gt_obj = pltpu.with_memory_space_constraint(gt_obj, pltpu.MemorySpace.HBM)
    op_obj = pltpu.with_memory_space_constraint(op_obj, pltpu.MemorySpace.HBM)
    bp_view = pltpu.with_memory_space_constraint(bp_view, pltpu.MemorySpace.HBM)
    bg_view = pltpu.with_memory_space_constraint(bg_view, pltpu.MemorySpace.HBM)
    gt4_view = pltpu.with_memory_space_constraint(gt4_view,
                                                  pltpu.MemorySpace.HBM)

    parts = _sc_bbox(bp_view, bg_view, gt4_view)
    bce, box_tc = _tc_main(op_obj, gt_obj, bp_view, bg_view)
    lam_o = jnp.asarray(lambda_rpn_objectness, jnp.float32).reshape(1, 1)
    lam_b = jnp.asarray(lambda_rpn_bbox, jnp.float32).reshape(1, 1)
    o1, o2, o3 = _combine(lam_o, lam_b, bce, box_tc, parts)
    return (o1.reshape(()), o2.reshape(()), o3.reshape(()))


# speedup vs baseline: 1.1739x; 1.0685x over previous
import functools

import jax
import jax.numpy as jnp
from jax import lax
from jax.experimental import pallas as pl
from jax.experimental.pallas import tpu as pltpu
from jax.experimental.pallas import tpu_sc as plsc

B, N = 32, 36864
BN = B * N
KBLK = N // 128
RPB = KBLK * 4
ROWS = B * RPB

TC_K = 144
TC_FRAC = TC_K * 4
GBLK = 2304


def _tc_body(op_ref, gt_ref, *refs):
    bp_refs = refs[:8]
    bg_refs = refs[8:16]
    bce_ref, box_ref = refs[16:]
    bb = pl.program_id(0)

    x = op_ref[...]
    tt = gt_ref[...]
    per = jnp.maximum(x, 0.0) - x * tt + jnp.log1p(jnp.exp(-jnp.abs(x)))
    s_bce = jnp.sum(per)

    maskc = (gt_ref[0:TC_K * 8, :] == 1.0).astype(jnp.float32)
    maskt = jnp.transpose(maskc.reshape(TC_K, 8, 128), (1, 0, 2))
    fk = jax.lax.broadcasted_iota(jnp.int32, (TC_K, TC_K * 4), 0)
    fr = jax.lax.broadcasted_iota(jnp.int32, (TC_K, TC_K * 4), 1)
    fold_m = (fr // 4 == fk).astype(jnp.float32)
    s_box = jnp.zeros((), jnp.float32)
    for j in range(8):
        d = bp_refs[j][...] - bg_refs[j][...]
        a = jnp.abs(d)
        c = jnp.minimum(a, 1.0)
        per2 = c * (a + a - c)
        folded = jax.lax.dot(fold_m, per2,
                             preferred_element_type=jnp.float32)
        s_box = s_box + jnp.sum(folded * maskt[j])

    @pl.when(bb == 0)
    def _():
        bce_ref[...] = jnp.zeros_like(bce_ref)
        box_ref[...] = jnp.zeros_like(box_ref)

    bce_ref[...] += jnp.broadcast_to(s_bce, (1, 1))
    box_ref[...] += jnp.broadcast_to(s_box, (1, 1))


def _tc_main(op4, gt4o, bpv, bgv):
    bbox_specs = [
        pl.BlockSpec((TC_FRAC, 128),
                     (lambda j: lambda bb: ((8 * bb + j) * (RPB // TC_FRAC),
                                            0))(j))
        for j in range(8)
    ]
    return pl.pallas_call(
        _tc_body,
        grid=(4,),
        in_specs=[
            pl.BlockSpec((GBLK, 128), lambda bb: (bb, 0)),
            pl.BlockSpec((GBLK, 128), lambda bb: (bb, 0)),
        ] + bbox_specs + bbox_specs,
        out_specs=[pl.BlockSpec((1, 1), lambda bb: (0, 0))] * 2,
        out_shape=[jax.ShapeDtypeStruct((1, 1), jnp.float32)] * 2,
    )(op4, gt4o, *([bpv] * 8), *([bgv] * 8))


NW = 32
CHUNK_ROWS = 144
SC_ROWS = RPB - TC_FRAC
N_CHUNKS = SC_ROWS // CHUNK_ROWS
SC_K = KBLK - TC_K

_sc_mesh = plsc.VectorSubcoreMesh(core_axis_name="c", subcore_axis_name="s")


@functools.partial(
    pl.kernel,
    mesh=_sc_mesh,
    out_type=jax.ShapeDtypeStruct((NW, 16), jnp.float32),
    scratch_types=[
        pltpu.VMEM((SC_K, 128), jnp.float32),
        pltpu.VMEM((2, CHUNK_ROWS, 128), jnp.float32),
        pltpu.VMEM((2, CHUNK_ROWS, 128), jnp.float32),
        pltpu.VMEM((16,), jnp.float32),
        pltpu.SemaphoreType.DMA,
        pltpu.SemaphoreType.DMA,
        pltpu.SemaphoreType.DMA,
    ],
)
def _sc_bbox(bp_hbm, bg_hbm, gt4_hbm, out_hbm,
             mask_v, bp_v, bg_v, acc_v, sem_m, sem_p, sem_g):
    cid = lax.axis_index("c")
    sid = lax.axis_index("s")
    w = sid * 2 + cid
    bb = w // 8
    bi = w % 8

    pltpu.sync_copy(gt4_hbm.at[bb, pl.ds(TC_K, SC_K), bi, :], mask_v)

    row0 = w * RPB + TC_FRAC

    def chunk_start(step, buf):
        r = row0 + step * CHUNK_ROWS
        cp = pltpu.async_copy(bp_hbm.at[pl.ds(r, CHUNK_ROWS), :],
                              bp_v.at[buf], sem_p)
        cg = pltpu.async_copy(bg_hbm.at[pl.ds(r, CHUNK_ROWS), :],
                              bg_v.at[buf], sem_g)
        return cp, cg

    def chunk_sum(step, buf):
        k0 = step * (CHUNK_ROWS // 4)

        def kk_body(kk, acc):
            def a_body(ai, acc2):
                a0 = ai * 16
                m = mask_v[k0 + kk, pl.ds(a0, 16)]

                def c_body(cc, acc3):
                    r = kk * 4 + cc
                    p = bp_v[buf, r, pl.ds(a0, 16)]
                    g = bg_v[buf, r, pl.ds(a0, 16)]
                    d = p - g
                    a = jnp.abs(d)
                    cl = jnp.minimum(a, 1.0)
                    per2 = cl * (a + a - cl)
                    return acc3 + per2 * m

                return lax.fori_loop(0, 4, c_body, acc2)

            return lax.fori_loop(0, 8, a_body, acc)

        return lax.fori_loop(0, CHUNK_ROWS // 4, kk_body,
                             jnp.zeros((16,), jnp.float32))

    total = jnp.zeros((16,), jnp.float32)
    cp, cg = chunk_start(0, 0)
    for step in range(N_CHUNKS):
        cp.wait()
        cg.wait()
        if step + 1 < N_CHUNKS:
            cp, cg = chunk_start(step + 1, (step + 1) % 2)
        total = total + chunk_sum(step, step % 2)

    acc_v[...] = total
    pltpu.sync_copy(acc_v, out_hbm.at[w])


def _combine_body(lo_ref, lb_ref, bce_ref, btc_ref, part_ref,
                  o1_ref, o2_ref, o3_ref):
    o1 = lo_ref[...] * bce_ref[...] * (1.0 / BN)
    o2 = lb_ref[...] * 0.5 * (jnp.broadcast_to(jnp.sum(part_ref[...]), (1, 1))
                              + btc_ref[...])
    o1_ref[...] = o1
    o2_ref[...] = o2
    o3_ref[...] = o1 + o2


def _combine(lam_o, lam_b, bce, box_tc, parts):
    return pl.pallas_call(
        _combine_body,
        in_specs=[pl.BlockSpec((1, 1), lambda: (0, 0))] * 4
        + [pl.BlockSpec((NW, 16), lambda: (0, 0))],
        out_specs=[pl.BlockSpec((1, 1), lambda: (0, 0))] * 3,
        out_shape=[jax.ShapeDtypeStruct((1, 1), jnp.float32)] * 3,
    )(lam_o, lam_b, bce, box_tc, parts)


def kernel(objectness_pred, bbox_pred, objectness_gt, bbox_gt,
           lambda_rpn_objectness, lambda_rpn_bbox):
    bp_view = (bbox_pred.reshape(B, KBLK, 128, 4)
               .transpose(0, 1, 3, 2).reshape(ROWS, 128))
    bg_view = (bbox_gt.reshape(B, KBLK, 128, 4)
               .transpose(0, 1, 3, 2).reshape(ROWS, 128))
    gt4_view = (objectness_gt.reshape(4, 8, KBLK, 128)
                .transpose(0, 2, 1, 3))
    gt_obj = gt4_view.reshape(B * KBLK, 128)
    op_obj = (objectness_pred.reshape(4, 8, KBLK, 128)
              .transpose(0, 2, 1, 3).reshape(B * KBLK, 128))
    gt_obj = pltpu.with_memory_space_constraint(gt_obj, pltpu.MemorySpace.HBM)
    op_obj = pltpu.with_memory_space_constraint(op_obj, pltpu.MemorySpace.HBM)
    bp_view = pltpu.with_memory_space_constraint(bp_view, pltpu.MemorySpace.HBM)
    bg_view = pltpu.with_memory_space_constraint(bg_view, pltpu.MemorySpace.HBM)
    gt4_view = pltpu.with_memory_space_constraint(gt4_view,
                                                  pltpu.MemorySpace.HBM)

    parts = _sc_bbox(bp_view, bg_view, gt4_view)
    bce, box_tc = _tc_main(op_obj, gt_obj, bp_view, bg_view)
    lam_o = jnp.asarray(lambda_rpn_objectness, jnp.float32).reshape(1, 1)
    lam_b = jnp.asarray(lambda_rpn_bbox, jnp.float32).reshape(1, 1)
    o1, o2, o3 = _combine(lam_o, lam_b, bce, box_tc, parts)
    return (o1.reshape(()), o2.reshape(()), o3.reshape(()))


# speedup vs baseline: 1.1880x; 1.0120x over previous
import functools

import jax
import jax.numpy as jnp
from jax import lax
from jax.experimental import pallas as pl
from jax.experimental.pallas import tpu as pltpu
from jax.experimental.pallas import tpu_sc as plsc

B, N = 32, 36864
BN = B * N
KBLK = N // 128
RPB = KBLK * 4
ROWS = B * RPB

TC_K = 144
TC_FRAC = TC_K * 4
GBLK = 2304


KSUB = TC_K // 2
BSUB = KSUB * 4
OBJ_BLK = 9216 // 8


def _tc_body(op_ref, gt_ref, gtm_ref, *refs):
    bp_refs = refs[:8]
    bg_refs = refs[8:16]
    bce_ref, box_ref = refs[16:]
    bb = pl.program_id(0)
    t = pl.program_id(1)

    x = op_ref[...]
    tt = gt_ref[...]
    per = jnp.maximum(x, 0.0) - x * tt + jnp.log1p(jnp.exp(-jnp.abs(x)))
    s_bce = jnp.sum(per)

    maskc = (gtm_ref[...] == 1.0).astype(jnp.float32)
    maskt = jnp.transpose(maskc.reshape(KSUB, 8, 128), (1, 0, 2))
    fk = jax.lax.broadcasted_iota(jnp.int32, (KSUB, BSUB), 0)
    fr = jax.lax.broadcasted_iota(jnp.int32, (KSUB, BSUB), 1)
    fold_m = (fr // 4 == fk).astype(jnp.float32)
    s_box = jnp.zeros((), jnp.float32)
    for j in range(8):
        d = bp_refs[j][...] - bg_refs[j][...]
        a = jnp.abs(d)
        c = jnp.minimum(a, 1.0)
        per2 = c * (a + a - c)
        folded = jax.lax.dot(fold_m, per2,
                             preferred_element_type=jnp.float32)
        s_box = s_box + jnp.sum(folded * maskt[j])

    @pl.when(jnp.logical_and(bb == 0, t == 0))
    def _():
        bce_ref[...] = jnp.zeros_like(bce_ref)
        box_ref[...] = jnp.zeros_like(box_ref)

    bce_ref[...] += jnp.broadcast_to(s_bce, (1, 1))
    box_ref[...] += jnp.broadcast_to(s_box, (1, 1))


def _tc_main(op4, gt4o, bpv, bgv):
    bbox_specs = [
        pl.BlockSpec((BSUB, 128),
                     (lambda j: lambda bb, t: (4 * (8 * bb + j) + t, 0))(j))
        for j in range(8)
    ]
    return pl.pallas_call(
        _tc_body,
        grid=(4, 2),
        in_specs=[
            pl.BlockSpec((OBJ_BLK, 128), lambda bb, t: (2 * bb + t, 0)),
            pl.BlockSpec((OBJ_BLK, 128), lambda bb, t: (2 * bb + t, 0)),
            pl.BlockSpec((KSUB * 8, 128), lambda bb, t: (4 * bb + t, 0)),
        ] + bbox_specs + bbox_specs,
        out_specs=[pl.BlockSpec((1, 1), lambda bb, t: (0, 0))] * 2,
        out_shape=[jax.ShapeDtypeStruct((1, 1), jnp.float32)] * 2,
    )(op4, gt4o, gt4o, *([bpv] * 8), *([bgv] * 8))


NW = 32
CHUNK_ROWS = 144
SC_ROWS = RPB - TC_FRAC
N_CHUNKS = SC_ROWS // CHUNK_ROWS
SC_K = KBLK - TC_K

_sc_mesh = plsc.VectorSubcoreMesh(core_axis_name="c", subcore_axis_name="s")


@functools.partial(
    pl.kernel,
    mesh=_sc_mesh,
    out_type=jax.ShapeDtypeStruct((NW, 16), jnp.float32),
    scratch_types=[
        pltpu.VMEM((SC_K, 128), jnp.float32),
        pltpu.VMEM((2, CHUNK_ROWS, 128), jnp.float32),
        pltpu.VMEM((2, CHUNK_ROWS, 128), jnp.float32),
        pltpu.VMEM((16,), jnp.float32),
        pltpu.SemaphoreType.DMA,
        pltpu.SemaphoreType.DMA,
        pltpu.SemaphoreType.DMA,
    ],
)
def _sc_bbox(bp_hbm, bg_hbm, gt4_hbm, out_hbm,
             mask_v, bp_v, bg_v, acc_v, sem_m, sem_p, sem_g):
    cid = lax.axis_index("c")
    sid = lax.axis_index("s")
    w = sid * 2 + cid
    bb = w // 8
    bi = w % 8

    pltpu.sync_copy(gt4_hbm.at[bb, pl.ds(TC_K, SC_K), bi, :], mask_v)

    row0 = w * RPB + TC_FRAC

    def chunk_start(step, buf):
        r = row0 + step * CHUNK_ROWS
        cp = pltpu.async_copy(bp_hbm.at[pl.ds(r, CHUNK_ROWS), :],
                              bp_v.at[buf], sem_p)
        cg = pltpu.async_copy(bg_hbm.at[pl.ds(r, CHUNK_ROWS), :],
                              bg_v.at[buf], sem_g)
        return cp, cg

    def chunk_sum(step, buf):
        k0 = step * (CHUNK_ROWS // 4)

        def kk_body(kk, acc):
            def a_body(ai, acc2):
                a0 = ai * 16
                m = mask_v[k0 + kk, pl.ds(a0, 16)]

                def c_body(cc, acc3):
                    r = kk * 4 + cc
                    p = bp_v[buf, r, pl.ds(a0, 16)]
                    g = bg_v[buf, r, pl.ds(a0, 16)]
                    d = p - g
                    a = jnp.abs(d)
                    cl = jnp.minimum(a, 1.0)
                    per2 = cl * (a + a - cl)
                    return acc3 + per2 * m

                return lax.fori_loop(0, 4, c_body, acc2)

            return lax.fori_loop(0, 8, a_body, acc)

        return lax.fori_loop(0, CHUNK_ROWS // 4, kk_body,
                             jnp.zeros((16,), jnp.float32))

    total = jnp.zeros((16,), jnp.float32)
    cp, cg = chunk_start(0, 0)
    for step in range(N_CHUNKS):
        cp.wait()
        cg.wait()
        if step + 1 < N_CHUNKS:
            cp, cg = chunk_start(step + 1, (step + 1) % 2)
        total = total + chunk_sum(step, step % 2)

    acc_v[...] = total
    pltpu.sync_copy(acc_v, out_hbm.at[w])


def _combine_body(lo_ref, lb_ref, bce_ref, btc_ref, part_ref,
                  o1_ref, o2_ref, o3_ref):
    o1 = lo_ref[...] * bce_ref[...] * (1.0 / BN)
    o2 = lb_ref[...] * 0.5 * (jnp.broadcast_to(jnp.sum(part_ref[...]), (1, 1))
                              + btc_ref[...])
    o1_ref[...] = o1
    o2_ref[...] = o2
    o3_ref[...] = o1 + o2


def _combine(lam_o, lam_b, bce, box_tc, parts):
    return pl.pallas_call(
        _combine_body,
        in_specs=[pl.BlockSpec((1, 1), lambda: (0, 0))] * 4
        + [pl.BlockSpec((NW, 16), lambda: (0, 0))],
        out_specs=[pl.BlockSpec((1, 1), lambda: (0, 0))] * 3,
        out_shape=[jax.ShapeDtypeStruct((1, 1), jnp.float32)] * 3,
    )(lam_o, lam_b, bce, box_tc, parts)


def kernel(objectness_pred, bbox_pred, objectness_gt, bbox_gt,
           lambda_rpn_objectness, lambda_rpn_bbox):
    bp_view = (bbox_pred.reshape(B, KBLK, 128, 4)
               .transpose(0, 1, 3, 2).reshape(ROWS, 128))
    bg_view = (bbox_gt.reshape(B, KBLK, 128, 4)
               .transpose(0, 1, 3, 2).reshape(ROWS, 128))
    gt4_view = (objectness_gt.reshape(4, 8, KBLK, 128)
                .transpose(0, 2, 1, 3))
    gt_obj = gt4_view.reshape(B * KBLK, 128)
    op_obj = (objectness_pred.reshape(4, 8, KBLK, 128)
              .transpose(0, 2, 1, 3).reshape(B * KBLK, 128))
    gt_obj = pltpu.with_memory_space_constraint(gt_obj, pltpu.MemorySpace.HBM)
    op_obj = pltpu.with_memory_space_constraint(op_obj, pltpu.MemorySpace.HBM)
    bp_view = pltpu.with_memory_space_constraint(bp_view, pltpu.MemorySpace.HBM)
    bg_view = pltpu.with_memory_space_constraint(bg_view, pltpu.MemorySpace.HBM)
    gt4_view = pltpu.with_memory_space_constraint(gt4_view,
                                                  pltpu.MemorySpace.HBM)

    parts = _sc_bbox(bp_view, bg_view, gt4_view)
    bce, box_tc = _tc_main(op_obj, gt_obj, bp_view, bg_view)
    lam_o = jnp.asarray(lambda_rpn_objectness, jnp.float32).reshape(1, 1)
    lam_b = jnp.asarray(lambda_rpn_bbox, jnp.float32).reshape(1, 1)
    o1, o2, o3 = _combine(lam_o, lam_b, bce, box_tc, parts)
    return (o1.reshape(()), o2.reshape(()), o3.reshape(()))
